# speedup vs baseline: 1.2307x; 1.2307x over previous
import functools

import jax
import jax.numpy as jnp
from jax import lax
from jax.experimental import pallas as pl
from jax.experimental.pallas import tpu as pltpu
from jax.experimental.pallas import tpu_sc as plsc

_NUM_CORES = 2
_NUM_SUBCORES = 16
_NW = _NUM_CORES * _NUM_SUBCORES

_CHUNK = 32


@functools.lru_cache(maxsize=None)
def _build_gather(n_rows: int, d_model: int):
    rows_per_worker = n_rows // _NW
    n_chunks = rows_per_worker // _CHUNK
    mesh = plsc.VectorSubcoreMesh(
        core_axis_name="c",
        subcore_axis_name="s",
        num_cores=_NUM_CORES,
        num_subcores=_NUM_SUBCORES,
    )

    @functools.partial(
        pl.kernel,
        mesh=mesh,
        out_type=jax.ShapeDtypeStruct((n_rows, d_model), jnp.float32),
        scratch_types=[
            pltpu.VMEM((n_chunks, _CHUNK), jnp.int32),
            pltpu.VMEM((2, _CHUNK, d_model), jnp.float32),
            pltpu.SemaphoreType.DMA,
        ],
    )
    def gather_kernel(idx_hbm, table_hbm, out_hbm, idx_v, rows_v, sem_g):
        wid = lax.axis_index("s") * _NUM_CORES + lax.axis_index("c")
        base = wid * rows_per_worker
        pltpu.sync_copy(idx_hbm.at[wid], idx_v)
        for c in range(n_chunks):
            buf = rows_v.at[c % 2]
            pltpu.async_copy(table_hbm.at[idx_v.at[c]], buf, sem_g).wait()
            pltpu.sync_copy(buf, out_hbm.at[pl.ds(base + c * _CHUNK, _CHUNK)])

    return gather_kernel


def kernel(input_ids, input_mask, table):
    del input_mask
    batch, seq = input_ids.shape
    vocab, d_model = table.shape
    n_rows = batch * seq
    ids_t = input_ids.T.reshape(_NW, -1, _CHUNK)
    out = _build_gather(n_rows, d_model)(ids_t, table)
    return out.reshape(seq, batch, d_model)


# speedup vs baseline: 1.3319x; 1.0823x over previous
import functools

import jax
import jax.numpy as jnp
from jax import lax
from jax.experimental import pallas as pl
from jax.experimental.pallas import tpu as pltpu
from jax.experimental.pallas import tpu_sc as plsc

_NUM_CORES = 2
_NUM_SUBCORES = 16
_NW = _NUM_CORES * _NUM_SUBCORES

_CHUNK = 32


@functools.lru_cache(maxsize=None)
def _build_gather(n_rows: int, d_model: int):
    rows_per_worker = n_rows // _NW
    n_chunks = rows_per_worker // _CHUNK
    mesh = plsc.VectorSubcoreMesh(
        core_axis_name="c",
        subcore_axis_name="s",
        num_cores=_NUM_CORES,
        num_subcores=_NUM_SUBCORES,
    )

    nbuf = 3

    @functools.partial(
        pl.kernel,
        mesh=mesh,
        out_type=jax.ShapeDtypeStruct((n_rows, d_model), jnp.float32),
        scratch_types=[
            pltpu.VMEM((n_chunks, _CHUNK), jnp.int32),
            pltpu.VMEM((nbuf, _CHUNK, d_model), jnp.float32),
            pltpu.SemaphoreType.DMA,
            pltpu.SemaphoreType.DMA,
        ],
    )
    def gather_kernel(idx_hbm, table_hbm, out_hbm, idx_v, rows_v, sem_g, sem_w):
        wid = lax.axis_index("s") * _NUM_CORES + lax.axis_index("c")
        base = wid * rows_per_worker
        pltpu.sync_copy(idx_hbm.at[wid], idx_v)

        def gather(c):
            return pltpu.async_copy(
                table_hbm.at[idx_v.at[c]], rows_v.at[c % nbuf], sem_g
            )

        def write(c):
            return pltpu.async_copy(
                rows_v.at[c % nbuf],
                out_hbm.at[pl.ds(base + c * _CHUNK, _CHUNK)],
                sem_w,
            )

        h_g = [None] * n_chunks
        h_w = [None] * n_chunks
        for c in range(nbuf - 1):
            h_g[c] = gather(c)
        for c in range(n_chunks):
            h_g[c].wait()
            h_w[c] = write(c)
            nxt = c + nbuf - 1
            if nxt < n_chunks:
                if c >= 1:
                    h_w[c - 1].wait()
                h_g[nxt] = gather(nxt)
        for c in range(n_chunks - nbuf, n_chunks):
            h_w[c].wait()

    return gather_kernel


def kernel(input_ids, input_mask, table):
    del input_mask
    batch, seq = input_ids.shape
    vocab, d_model = table.shape
    n_rows = batch * seq
    ids_t = input_ids.T.reshape(_NW, -1, _CHUNK)
    out = _build_gather(n_rows, d_model)(ids_t, table)
    return out.reshape(seq, batch, d_model)


# speedup vs baseline: 2.2202x; 1.6669x over previous
import functools

import jax
import jax.numpy as jnp
from jax import lax
from jax.experimental import pallas as pl
from jax.experimental.pallas import tpu as pltpu
from jax.experimental.pallas import tpu_sc as plsc

_NUM_CORES = 2
_NUM_SUBCORES = 16
_NW = _NUM_CORES * _NUM_SUBCORES


@functools.lru_cache(maxsize=None)
def _build_gather(seq: int, batch: int, d_model: int):
    s_per_w = seq // _NW
    mesh = plsc.VectorSubcoreMesh(
        core_axis_name="c",
        subcore_axis_name="s",
        num_cores=_NUM_CORES,
        num_subcores=_NUM_SUBCORES,
    )

    s_grp = 8
    n_grp = s_per_w // s_grp

    @functools.partial(
        pl.kernel,
        mesh=mesh,
        out_type=jax.ShapeDtypeStruct((seq, batch, d_model), jnp.float32),
        scratch_types=[
            pltpu.VMEM((s_per_w * batch,), jnp.int32),
            pltpu.VMEM((s_grp * batch, d_model), jnp.float32),
            pltpu.SemaphoreType.DMA,
        ],
    )
    def gather_kernel(idx_hbm, table_hbm, out_hbm, idx_v, rows_v, sem_g):
        wid = lax.axis_index("s") * _NUM_CORES + lax.axis_index("c")
        s_base = wid * s_per_w
        pltpu.sync_copy(idx_hbm.at[wid], idx_v)

        @pl.loop(0, n_grp)
        def _(g):
            pltpu.async_copy(
                table_hbm.at[idx_v.at[pl.ds(g * s_grp * batch, s_grp * batch)]],
                rows_v,
                sem_g,
            ).wait()
            for i in range(s_grp):
                pltpu.sync_copy(
                    rows_v.at[pl.ds(i * batch, batch)],
                    out_hbm.at[s_base + g * s_grp + i],
                )

    return gather_kernel


def kernel(input_ids, input_mask, table):
    del input_mask
    batch, seq = input_ids.shape
    _, d_model = table.shape
    ids_t = input_ids.T.reshape(_NW, -1)
    return _build_gather(seq, batch, d_model)(ids_t, table)


# speedup vs baseline: 2.2295x; 1.0042x over previous
import functools

import jax
import jax.numpy as jnp
from jax import lax
from jax.experimental import pallas as pl
from jax.experimental.pallas import tpu as pltpu
from jax.experimental.pallas import tpu_sc as plsc

_NUM_CORES = 2
_NUM_SUBCORES = 16
_NW = _NUM_CORES * _NUM_SUBCORES


@functools.lru_cache(maxsize=None)
def _build_gather(seq: int, batch: int, d_model: int):
    s_per_w = seq // _NW
    mesh = plsc.VectorSubcoreMesh(
        core_axis_name="c",
        subcore_axis_name="s",
        num_cores=_NUM_CORES,
        num_subcores=_NUM_SUBCORES,
    )

    s_grp = 8
    n_grp = s_per_w // s_grp

    nbuf = 2

    @functools.partial(
        pl.kernel,
        mesh=mesh,
        out_type=jax.ShapeDtypeStruct((seq, batch, d_model), jnp.float32),
        scratch_types=[
            pltpu.VMEM((s_per_w * batch,), jnp.int32),
            pltpu.VMEM((nbuf, s_grp * batch, d_model), jnp.float32),
            pltpu.SemaphoreType.DMA,
            pltpu.SemaphoreType.DMA,
        ],
    )
    def gather_kernel(idx_hbm, table_hbm, out_hbm, idx_v, rows_v, sem_g, sem_w):
        wid = lax.axis_index("s") * _NUM_CORES + lax.axis_index("c")
        s_base = wid * s_per_w
        pltpu.sync_copy(idx_hbm.at[wid], idx_v)

        def gather(g):
            return pltpu.async_copy(
                table_hbm.at[idx_v.at[pl.ds(g * s_grp * batch, s_grp * batch)]],
                rows_v.at[g % nbuf],
                sem_g,
            )

        def writes(g):
            return [
                pltpu.async_copy(
                    rows_v.at[g % nbuf].at[pl.ds(i * batch, batch)],
                    out_hbm.at[s_base + g * s_grp + i],
                    sem_w,
                )
                for i in range(s_grp)
            ]

        h_g = [None] * n_grp
        h_w = [None] * n_grp
        for g in range(nbuf - 1):
            h_g[g] = gather(g)
        for g in range(n_grp):
            h_g[g].wait()
            h_w[g] = writes(g)
            nxt = g + nbuf - 1
            if nxt < n_grp:
                if g >= 1:
                    for h in h_w[g - 1]:
                        h.wait()
                h_g[nxt] = gather(nxt)
        for g in range(n_grp - nbuf, n_grp):
            for h in h_w[g]:
                h.wait()

    return gather_kernel


def kernel(input_ids, input_mask, table):
    del input_mask
    batch, seq = input_ids.shape
    _, d_model = table.shape
    ids_t = input_ids.T.reshape(_NW, -1)
    return _build_gather(seq, batch, d_model)(ids_t, table)


# speedup vs baseline: 2.5479x; 1.1428x over previous
import functools

import jax
import jax.numpy as jnp
from jax import lax
from jax.experimental import pallas as pl
from jax.experimental.pallas import tpu as pltpu
from jax.experimental.pallas import tpu_sc as plsc

_NUM_CORES = 2
_NUM_SUBCORES = 16
_NW = _NUM_CORES * _NUM_SUBCORES


@functools.lru_cache(maxsize=None)
def _build_gather(seq: int, batch: int, d_model: int):
    s_per_w = seq // _NW
    s_grp = 8
    n_grp = s_per_w // s_grp
    nbuf = 3

    mesh = plsc.VectorSubcoreMesh(
        core_axis_name="c",
        subcore_axis_name="s",
        num_cores=_NUM_CORES,
        num_subcores=_NUM_SUBCORES,
    )

    @functools.partial(
        pl.kernel,
        mesh=mesh,
        out_type=jax.ShapeDtypeStruct((seq, batch, d_model), jnp.float32),
        scratch_types=[
            pltpu.VMEM((s_per_w * batch,), jnp.int32),
            pltpu.VMEM((nbuf, s_grp * batch, d_model), jnp.float32),
            pltpu.SemaphoreType.DMA,
            pltpu.SemaphoreType.DMA,
        ],
    )
    def gather_kernel(idx_hbm, table_hbm, out_hbm, idx_v, rows_v, sem_g, sem_w):
        wid = lax.axis_index("s") * _NUM_CORES + lax.axis_index("c")
        s_base = wid * s_per_w
        pltpu.sync_copy(idx_hbm.at[wid], idx_v)

        def gather(g):
            return pltpu.async_copy(
                table_hbm.at[idx_v.at[pl.ds(g * s_grp * batch, s_grp * batch)]],
                rows_v.at[g % nbuf],
                sem_g,
            )

        def write(g):
            buf = rows_v.at[g % nbuf]
            return [
                pltpu.async_copy(
                    buf.at[pl.ds(i * batch, batch)],
                    out_hbm.at[s_base + g * s_grp + i],
                    sem_w,
                )
                for i in range(s_grp)
            ]

        h_g = [None] * n_grp
        h_w = [None] * n_grp
        for g in range(nbuf - 1):
            h_g[g] = gather(g)
        for g in range(n_grp):
            h_g[g].wait()
            h_w[g] = write(g)
            nxt = g + nbuf - 1
            if nxt < n_grp:
                if g >= 1:
                    for h in h_w[g - 1]:
                        h.wait()
                h_g[nxt] = gather(nxt)
        for g in range(n_grp - nbuf, n_grp):
            for h in h_w[g]:
                h.wait()

    return gather_kernel


def kernel(input_ids, input_mask, table):
    del input_mask
    batch, seq = input_ids.shape
    _, d_model = table.shape
    ids_t = input_ids.T.reshape(_NW, -1)
    return _build_gather(seq, batch, d_model)(ids_t, table)
